# baseline (speedup 1.0000x reference)
import functools

import jax
import jax.numpy as jnp
from jax import lax
from jax.experimental import pallas as pl
from jax.experimental.pallas import tpu as pltpu
from jax.experimental.pallas import tpu_sc as plsc

B = 16384
D_USER, D_COUNTRY, D_DEVICE, D_DENSE = 32, 16, 16, 128
V_USER = 1000000
V_SMALL = 1000
TOTAL = D_USER + D_COUNTRY + D_DEVICE + D_DENSE
OUT_D = 128
EPS = 1.1920928955078125e-07

LANE_TILES = (V_USER + 127) // 128
_F_OFF = [(f // 8) * (LANE_TILES * 1024) + (f % 8) * 128 for f in range(D_USER)]


def _sc_gather_user(user_id, emb_user_t):
    info = plsc.get_sparse_core_info()
    nw = info.num_cores * info.num_subcores
    bpw = B // nw
    npe = D_USER * bpw
    mesh = plsc.VectorSubcoreMesh(core_axis_name="c", subcore_axis_name="s")

    nbuf = 8

    @functools.partial(
        pl.kernel,
        out_type=jax.ShapeDtypeStruct((D_USER, B), jnp.float32),
        mesh=mesh,
        compiler_params=pltpu.CompilerParams(needs_layout_passes=False),
        scratch_types=[
            pltpu.VMEM((bpw,), jnp.int32),
            pltpu.VMEM((npe,), jnp.float32),
        ] + [pltpu.VMEM((D_USER, 128), jnp.float32) for _ in range(nbuf)]
          + [pltpu.SemaphoreType.DMA],
    )
    def gather_kernel(uid_h, tu_h, out_h, uidx, vals, *rest):
        slabs, sem = list(rest[:nbuf]), rest[nbuf]
        wid = lax.axis_index("s") * info.num_cores + lax.axis_index("c")
        base = pl.multiple_of(wid * bpw, bpw)
        pltpu.sync_copy(uid_h.at[pl.ds(base, bpw)], uidx)
        iota16 = lax.iota(jnp.int32, 16)

        def extract(slab, lane_scalar, row):
            lane = jnp.full((16,), lane_scalar, jnp.int32)
            x0 = plsc.load_gather(slab, [iota16, lane])
            x1 = plsc.load_gather(slab, [iota16 + 16, lane])
            idx0 = iota16 * bpw + row
            plsc.store_scatter(vals, [idx0], x0)
            plsc.store_scatter(vals, [idx0 + 16 * bpw], x1)

        def chunk_body(c, carry):
            r0 = pl.multiple_of(c * 16, 16)
            v16 = uidx[pl.ds(r0, 16)]
            t16 = (v16 >> 7) << 7
            l16 = v16 & 127
            copies = []
            for j in range(16):
                off = pl.multiple_of(t16[j], 128)
                copies.append(pltpu.async_copy(
                    tu_h.at[:, pl.ds(off, 128)], slabs[j % nbuf], sem))
                if j >= nbuf:
                    copies[j - nbuf].wait()
                    extract(slabs[(j - nbuf) % nbuf], l16[j - nbuf],
                            r0 + (j - nbuf))
            for j in range(16 - nbuf, 16):
                copies[j].wait()
                extract(slabs[j % nbuf], l16[j], r0 + j)
            return carry

        lax.fori_loop(0, bpw // 16, chunk_body, 0, unroll=False)
        for f in range(D_USER):
            pltpu.sync_copy(vals.at[pl.ds(f * bpw, bpw)],
                            out_h.at[f, pl.ds(base, bpw)])

    return gather_kernel(user_id, emb_user_t)


def _tc_body(eut_ref, cid_ref, did_ref, dp_ref, tc_ref, td_ref, w_ref, b_ref,
             out_ref):
    eut = eut_ref[...]
    dp = dp_ref[...]
    lanes = lax.broadcasted_iota(jnp.int32, (1, V_SMALL), 1)
    onehot_c = (cid_ref[...] == lanes).astype(jnp.float32)
    onehot_d = (did_ref[...] == lanes).astype(jnp.float32)
    ec = jnp.dot(onehot_c, tc_ref[...], preferred_element_type=jnp.float32)
    ed = jnp.dot(onehot_d, td_ref[...], preferred_element_type=jnp.float32)
    ones_u = jnp.ones((D_USER, 1), jnp.float32)
    ssq = (lax.dot_general(eut * eut, ones_u, (((0,), (0,)), ((), ())),
                           preferred_element_type=jnp.float32)
           + jnp.sum(ec * ec, axis=1, keepdims=True)
           + jnp.sum(ed * ed, axis=1, keepdims=True)
           + jnp.sum(dp * dp, axis=1, keepdims=True))
    scale = lax.rsqrt(ssq * (1.0 / TOTAL) + EPS)
    acc = lax.dot_general(eut, w_ref[0:D_USER, :], (((0,), (0,)), ((), ())),
                          preferred_element_type=jnp.float32)
    acc += jnp.dot(ec, w_ref[D_USER:D_USER + D_COUNTRY, :],
                   preferred_element_type=jnp.float32)
    acc += jnp.dot(ed, w_ref[D_USER + D_COUNTRY:D_USER + D_COUNTRY + D_DEVICE, :],
                   preferred_element_type=jnp.float32)
    acc += jnp.dot(dp, w_ref[TOTAL - D_DENSE:TOTAL, :],
                   preferred_element_type=jnp.float32)
    out_ref[...] = scale * acc + b_ref[...]


def _tc_norm_matmul(eut, cid, did, dp, tbl_c, tbl_d, w, b):
    blk = 2048
    grid = (B // blk,)
    return pl.pallas_call(
        _tc_body,
        grid=grid,
        in_specs=[
            pl.BlockSpec((D_USER, blk), lambda i: (0, i)),
            pl.BlockSpec((blk, 1), lambda i: (i, 0)),
            pl.BlockSpec((blk, 1), lambda i: (i, 0)),
            pl.BlockSpec((blk, D_DENSE), lambda i: (i, 0)),
            pl.BlockSpec((V_SMALL, D_COUNTRY), lambda i: (0, 0)),
            pl.BlockSpec((V_SMALL, D_DEVICE), lambda i: (0, 0)),
            pl.BlockSpec((TOTAL, OUT_D), lambda i: (0, 0)),
            pl.BlockSpec((1, OUT_D), lambda i: (0, 0)),
        ],
        out_specs=pl.BlockSpec((blk, OUT_D), lambda i: (i, 0)),
        out_shape=jax.ShapeDtypeStruct((B, OUT_D), jnp.float32),
    )(eut, cid, did, dp, tbl_c, tbl_d, w, b)


def kernel(user_id, country, device, dense_profile, emb_user, emb_country,
           emb_device, rms_weight, W, b):
    eut = _sc_gather_user(user_id.astype(jnp.int32), emb_user.T)
    w_scaled = rms_weight[:, None] * W
    return _tc_norm_matmul(eut,
                           country.astype(jnp.int32).reshape(B, 1),
                           device.astype(jnp.int32).reshape(B, 1),
                           dense_profile, emb_country, emb_device,
                           w_scaled, b.reshape(1, OUT_D))


# speedup vs baseline: 1.0462x; 1.0462x over previous
import functools

import jax
import jax.numpy as jnp
from jax import lax
from jax.experimental import pallas as pl
from jax.experimental.pallas import tpu as pltpu
from jax.experimental.pallas import tpu_sc as plsc

B = 16384
D_USER, D_COUNTRY, D_DEVICE, D_DENSE = 32, 16, 16, 128
V_USER = 1000000
V_SMALL = 1000
TOTAL = D_USER + D_COUNTRY + D_DEVICE + D_DENSE
OUT_D = 128
EPS = 1.1920928955078125e-07

LANE_TILES = (V_USER + 127) // 128
_F_OFF = [(f // 8) * (LANE_TILES * 1024) + (f % 8) * 128 for f in range(D_USER)]


def _sc_gather_user(user_id, emb_user_t):
    info = plsc.get_sparse_core_info()
    nw = info.num_cores * info.num_subcores
    bpw = B // nw
    npe = D_USER * bpw
    mesh = plsc.VectorSubcoreMesh(core_axis_name="c", subcore_axis_name="s")

    nbuf = 16
    cw = 32

    @functools.partial(
        pl.kernel,
        out_type=jax.ShapeDtypeStruct((D_USER, B), jnp.float32),
        mesh=mesh,
        compiler_params=pltpu.CompilerParams(needs_layout_passes=False),
        scratch_types=[
            pltpu.VMEM((bpw,), jnp.int32),
            pltpu.VMEM((npe,), jnp.float32),
        ] + [pltpu.VMEM((D_USER, 128), jnp.float32) for _ in range(nbuf)]
          + [pltpu.SemaphoreType.DMA],
    )
    def gather_kernel(uid_h, tu_h, out_h, uidx, vals, *rest):
        slabs, sem = list(rest[:nbuf]), rest[nbuf]
        wid = lax.axis_index("s") * info.num_cores + lax.axis_index("c")
        base = pl.multiple_of(wid * bpw, bpw)
        pltpu.sync_copy(uid_h.at[pl.ds(base, bpw)], uidx)
        iota16 = lax.iota(jnp.int32, 16)

        def extract(slab, lane_scalar, row):
            lane = jnp.full((16,), lane_scalar, jnp.int32)
            x0 = plsc.load_gather(slab, [iota16, lane])
            x1 = plsc.load_gather(slab, [iota16 + 16, lane])
            idx0 = iota16 * bpw + row
            plsc.store_scatter(vals, [idx0], x0)
            plsc.store_scatter(vals, [idx0 + 16 * bpw], x1)

        def chunk_body(c, carry):
            r0 = pl.multiple_of(c * cw, cw)
            tv = []
            for h in range(cw // 16):
                v16 = uidx[pl.ds(r0 + h * 16, 16)]
                tv.append(((v16 >> 7) << 7, v16 & 127))
            copies = []
            for j in range(cw):
                off = pl.multiple_of(tv[j // 16][0][j % 16], 128)
                copies.append(pltpu.async_copy(
                    tu_h.at[:, pl.ds(off, 128)], slabs[j % nbuf], sem))
                if j >= nbuf:
                    k = j - nbuf
                    copies[k].wait()
                    extract(slabs[k % nbuf], tv[k // 16][1][k % 16], r0 + k)
            for j in range(cw - nbuf, cw):
                copies[j].wait()
                extract(slabs[j % nbuf], tv[j // 16][1][j % 16], r0 + j)
            return carry

        lax.fori_loop(0, bpw // cw, chunk_body, 0, unroll=False)
        for f in range(D_USER):
            pltpu.sync_copy(vals.at[pl.ds(f * bpw, bpw)],
                            out_h.at[f, pl.ds(base, bpw)])

    return gather_kernel(user_id, emb_user_t)


def _tc_body(eut_ref, cid_ref, did_ref, dp_ref, tc_ref, td_ref, w_ref, b_ref,
             out_ref):
    eut = eut_ref[...]
    dp = dp_ref[...]
    lanes = lax.broadcasted_iota(jnp.int32, (1, V_SMALL), 1)
    onehot_c = (cid_ref[...] == lanes).astype(jnp.float32)
    onehot_d = (did_ref[...] == lanes).astype(jnp.float32)
    ec = jnp.dot(onehot_c, tc_ref[...], preferred_element_type=jnp.float32)
    ed = jnp.dot(onehot_d, td_ref[...], preferred_element_type=jnp.float32)
    ones_u = jnp.ones((D_USER, 1), jnp.float32)
    ssq = (lax.dot_general(eut * eut, ones_u, (((0,), (0,)), ((), ())),
                           preferred_element_type=jnp.float32)
           + jnp.sum(ec * ec, axis=1, keepdims=True)
           + jnp.sum(ed * ed, axis=1, keepdims=True)
           + jnp.sum(dp * dp, axis=1, keepdims=True))
    scale = lax.rsqrt(ssq * (1.0 / TOTAL) + EPS)
    acc = lax.dot_general(eut, w_ref[0:D_USER, :], (((0,), (0,)), ((), ())),
                          preferred_element_type=jnp.float32)
    acc += jnp.dot(ec, w_ref[D_USER:D_USER + D_COUNTRY, :],
                   preferred_element_type=jnp.float32)
    acc += jnp.dot(ed, w_ref[D_USER + D_COUNTRY:D_USER + D_COUNTRY + D_DEVICE, :],
                   preferred_element_type=jnp.float32)
    acc += jnp.dot(dp, w_ref[TOTAL - D_DENSE:TOTAL, :],
                   preferred_element_type=jnp.float32)
    out_ref[...] = scale * acc + b_ref[...]


def _tc_norm_matmul(eut, cid, did, dp, tbl_c, tbl_d, w, b):
    blk = 4096
    grid = (B // blk,)
    return pl.pallas_call(
        _tc_body,
        grid=grid,
        in_specs=[
            pl.BlockSpec((D_USER, blk), lambda i: (0, i)),
            pl.BlockSpec((blk, 1), lambda i: (i, 0)),
            pl.BlockSpec((blk, 1), lambda i: (i, 0)),
            pl.BlockSpec((blk, D_DENSE), lambda i: (i, 0)),
            pl.BlockSpec((V_SMALL, D_COUNTRY), lambda i: (0, 0)),
            pl.BlockSpec((V_SMALL, D_DEVICE), lambda i: (0, 0)),
            pl.BlockSpec((TOTAL, OUT_D), lambda i: (0, 0)),
            pl.BlockSpec((1, OUT_D), lambda i: (0, 0)),
        ],
        out_specs=pl.BlockSpec((blk, OUT_D), lambda i: (i, 0)),
        out_shape=jax.ShapeDtypeStruct((B, OUT_D), jnp.float32),
    )(eut, cid, did, dp, tbl_c, tbl_d, w, b)


def kernel(user_id, country, device, dense_profile, emb_user, emb_country,
           emb_device, rms_weight, W, b):
    eut = _sc_gather_user(user_id.astype(jnp.int32), emb_user.T)
    w_scaled = rms_weight[:, None] * W
    return _tc_norm_matmul(eut,
                           country.astype(jnp.int32).reshape(B, 1),
                           device.astype(jnp.int32).reshape(B, 1),
                           dense_profile, emb_country, emb_device,
                           w_scaled, b.reshape(1, OUT_D))


# speedup vs baseline: 1.1094x; 1.0604x over previous
import functools

import jax
import jax.numpy as jnp
from jax import lax
from jax.experimental import pallas as pl
from jax.experimental.pallas import tpu as pltpu
from jax.experimental.pallas import tpu_sc as plsc

B = 16384
D_USER, D_COUNTRY, D_DEVICE, D_DENSE = 32, 16, 16, 128
V_USER = 1000000
V_SMALL = 1000
TOTAL = D_USER + D_COUNTRY + D_DEVICE + D_DENSE
OUT_D = 128
EPS = 1.1920928955078125e-07

LANE_TILES = (V_USER + 127) // 128
_F_OFF = [(f // 8) * (LANE_TILES * 1024) + (f % 8) * 128 for f in range(D_USER)]


def _sc_gather_user(user_id, emb_user_t):
    info = plsc.get_sparse_core_info()
    nw = info.num_cores * info.num_subcores
    bpw = B // nw
    npe = D_USER * bpw
    mesh = plsc.VectorSubcoreMesh(core_axis_name="c", subcore_axis_name="s")

    nbuf = 16
    cw = 32

    @functools.partial(
        pl.kernel,
        out_type=jax.ShapeDtypeStruct((D_USER, B), jnp.float32),
        mesh=mesh,
        compiler_params=pltpu.CompilerParams(needs_layout_passes=False),
        scratch_types=[
            pltpu.VMEM((bpw,), jnp.int32),
            pltpu.VMEM((npe,), jnp.float32),
        ] + [pltpu.VMEM((D_USER, 128), jnp.float32) for _ in range(nbuf)]
          + [pltpu.SemaphoreType.DMA],
    )
    def gather_kernel(uid_h, tu_h, out_h, uidx, vals, *rest):
        slabs, sem = list(rest[:nbuf]), rest[nbuf]
        wid = lax.axis_index("s") * info.num_cores + lax.axis_index("c")
        base = pl.multiple_of(wid * bpw, bpw)
        pltpu.sync_copy(uid_h.at[pl.ds(base, bpw)], uidx)
        iota16 = lax.iota(jnp.int32, 16)

        def extract(slab, lane_scalar, row):
            lane = jnp.full((16,), lane_scalar, jnp.int32)
            x0 = plsc.load_gather(slab, [iota16, lane])
            x1 = plsc.load_gather(slab, [iota16 + 16, lane])
            idx0 = iota16 * bpw + row
            plsc.store_scatter(vals, [idx0], x0)
            plsc.store_scatter(vals, [idx0 + 16 * bpw], x1)

        def chunk_body(c, carry):
            r0 = pl.multiple_of(c * cw, cw)
            tv = []
            for h in range(cw // 16):
                v16 = uidx[pl.ds(r0 + h * 16, 16)]
                tv.append(((v16 >> 7) << 7, v16 & 127))
            copies = []
            for j in range(cw):
                off = pl.multiple_of(tv[j // 16][0][j % 16], 128)
                copies.append(pltpu.async_copy(
                    tu_h.at[:, pl.ds(off, 128)], slabs[j % nbuf], sem))
                if j >= nbuf:
                    k = j - nbuf
                    copies[k].wait()
                    extract(slabs[k % nbuf], tv[k // 16][1][k % 16], r0 + k)
            for j in range(cw - nbuf, cw):
                copies[j].wait()
                extract(slabs[j % nbuf], tv[j // 16][1][j % 16], r0 + j)
            return carry

        lax.fori_loop(0, bpw // cw, chunk_body, 0, unroll=False)
        for f in range(D_USER):
            pltpu.sync_copy(vals.at[pl.ds(f * bpw, bpw)],
                            out_h.at[f, pl.ds(base, bpw)])

    return gather_kernel(user_id, emb_user_t)


def _tc_rest_body(cid_ref, did_ref, dp_ref, tc_ref, td_ref, w_ref,
                  acc_ref, ssq_ref):
    dp = dp_ref[...]
    lanes = lax.broadcasted_iota(jnp.int32, (1, V_SMALL), 1)
    onehot_c = (cid_ref[...] == lanes).astype(jnp.float32)
    onehot_d = (did_ref[...] == lanes).astype(jnp.float32)
    ec = jnp.dot(onehot_c, tc_ref[...], preferred_element_type=jnp.float32)
    ed = jnp.dot(onehot_d, td_ref[...], preferred_element_type=jnp.float32)
    ssq_ref[...] = (jnp.sum(ec * ec, axis=1, keepdims=True)
                    + jnp.sum(ed * ed, axis=1, keepdims=True)
                    + jnp.sum(dp * dp, axis=1, keepdims=True))
    acc = jnp.dot(ec, w_ref[D_USER:D_USER + D_COUNTRY, :],
                  preferred_element_type=jnp.float32)
    acc += jnp.dot(ed, w_ref[D_USER + D_COUNTRY:D_USER + D_COUNTRY + D_DEVICE, :],
                   preferred_element_type=jnp.float32)
    acc += jnp.dot(dp, w_ref[TOTAL - D_DENSE:TOTAL, :],
                   preferred_element_type=jnp.float32)
    acc_ref[...] = acc


def _tc_rest(cid, did, dp, tbl_c, tbl_d, w):
    blk = 4096
    grid = (B // blk,)
    return pl.pallas_call(
        _tc_rest_body,
        grid=grid,
        in_specs=[
            pl.BlockSpec((blk, 1), lambda i: (i, 0)),
            pl.BlockSpec((blk, 1), lambda i: (i, 0)),
            pl.BlockSpec((blk, D_DENSE), lambda i: (i, 0)),
            pl.BlockSpec((V_SMALL, D_COUNTRY), lambda i: (0, 0)),
            pl.BlockSpec((V_SMALL, D_DEVICE), lambda i: (0, 0)),
            pl.BlockSpec((TOTAL, OUT_D), lambda i: (0, 0)),
        ],
        out_specs=[
            pl.BlockSpec((blk, OUT_D), lambda i: (i, 0)),
            pl.BlockSpec((blk, 1), lambda i: (i, 0)),
        ],
        out_shape=[
            jax.ShapeDtypeStruct((B, OUT_D), jnp.float32),
            jax.ShapeDtypeStruct((B, 1), jnp.float32),
        ],
    )(cid, did, dp, tbl_c, tbl_d, w)


def _tc_combine_body(eut_ref, accr_ref, ssqr_ref, w_ref, b_ref, out_ref):
    eut = eut_ref[...]
    ones_u = jnp.ones((D_USER, 1), jnp.float32)
    ssq = ssqr_ref[...] + lax.dot_general(
        eut * eut, ones_u, (((0,), (0,)), ((), ())),
        preferred_element_type=jnp.float32)
    scale = lax.rsqrt(ssq * (1.0 / TOTAL) + EPS)
    acc = accr_ref[...] + lax.dot_general(
        eut, w_ref[0:D_USER, :], (((0,), (0,)), ((), ())),
        preferred_element_type=jnp.float32)
    out_ref[...] = scale * acc + b_ref[...]


def _tc_combine(eut, accr, ssqr, w, b):
    blk = 4096
    grid = (B // blk,)
    return pl.pallas_call(
        _tc_combine_body,
        grid=grid,
        in_specs=[
            pl.BlockSpec((D_USER, blk), lambda i: (0, i)),
            pl.BlockSpec((blk, OUT_D), lambda i: (i, 0)),
            pl.BlockSpec((blk, 1), lambda i: (i, 0)),
            pl.BlockSpec((TOTAL, OUT_D), lambda i: (0, 0)),
            pl.BlockSpec((1, OUT_D), lambda i: (0, 0)),
        ],
        out_specs=pl.BlockSpec((blk, OUT_D), lambda i: (i, 0)),
        out_shape=jax.ShapeDtypeStruct((B, OUT_D), jnp.float32),
    )(eut, accr, ssqr, w, b)


def kernel(user_id, country, device, dense_profile, emb_user, emb_country,
           emb_device, rms_weight, W, b):
    eut = _sc_gather_user(user_id.astype(jnp.int32), emb_user.T)
    w_scaled = rms_weight[:, None] * W
    accr, ssqr = _tc_rest(country.astype(jnp.int32).reshape(B, 1),
                          device.astype(jnp.int32).reshape(B, 1),
                          dense_profile, emb_country, emb_device, w_scaled)
    return _tc_combine(eut, accr, ssqr, w_scaled, b.reshape(1, OUT_D))


# speedup vs baseline: 1.1733x; 1.0576x over previous
import functools

import jax
import jax.numpy as jnp
from jax import lax
from jax.experimental import pallas as pl
from jax.experimental.pallas import tpu as pltpu
from jax.experimental.pallas import tpu_sc as plsc

B = 16384
D_USER, D_COUNTRY, D_DEVICE, D_DENSE = 32, 16, 16, 128
V_USER = 1000000
V_SMALL = 1000
TOTAL = D_USER + D_COUNTRY + D_DEVICE + D_DENSE
OUT_D = 128
EPS = 1.1920928955078125e-07

LANE_TILES = (V_USER + 127) // 128
_F_OFF = [(f // 8) * (LANE_TILES * 1024) + (f % 8) * 128 for f in range(D_USER)]


def _sc_gather_user(user_id, emb_user_t):
    info = plsc.get_sparse_core_info()
    nw = info.num_cores * info.num_subcores
    bpw = B // nw
    npe = D_USER * bpw
    mesh = plsc.VectorSubcoreMesh(core_axis_name="c", subcore_axis_name="s")

    nbuf = 16
    cw = 32

    @functools.partial(
        pl.kernel,
        out_type=jax.ShapeDtypeStruct((D_USER, B), jnp.float32),
        mesh=mesh,
        compiler_params=pltpu.CompilerParams(needs_layout_passes=False),
        scratch_types=[
            pltpu.VMEM((bpw,), jnp.int32),
            pltpu.VMEM((npe,), jnp.float32),
        ] + [pltpu.VMEM((D_USER, 128), jnp.float32) for _ in range(nbuf)]
          + [pltpu.SemaphoreType.DMA],
    )
    def gather_kernel(uid_h, tu_h, out_h, uidx, vals, *rest):
        slabs, sem = list(rest[:nbuf]), rest[nbuf]
        wid = lax.axis_index("s") * info.num_cores + lax.axis_index("c")
        base = pl.multiple_of(wid * bpw, bpw)
        pltpu.sync_copy(uid_h.at[pl.ds(base, bpw)], uidx)
        iota16 = lax.iota(jnp.int32, 16)

        def extract(slab, lane_scalar, row):
            lane = jnp.full((16,), lane_scalar, jnp.int32)
            x0 = plsc.load_gather(slab, [iota16, lane])
            x1 = plsc.load_gather(slab, [iota16 + 16, lane])
            idx0 = iota16 * bpw + row
            plsc.store_scatter(vals, [idx0], x0)
            plsc.store_scatter(vals, [idx0 + 16 * bpw], x1)

        ngrp = bpw // 16

        def issue_group(g):
            v16 = uidx[pl.ds(pl.multiple_of(g * 16, 16), 16)]
            t16 = (v16 >> 7) << 7
            for j in range(16):
                off = pl.multiple_of(t16[j], 128)
                pltpu.async_copy(tu_h.at[:, pl.ds(off, 128)], slabs[j], sem)

        issue_group(0)

        def grp_body(g, carry):
            r0 = pl.multiple_of(g * 16, 16)
            l16 = uidx[pl.ds(r0, 16)] & 127
            gn = pl.multiple_of(((g + 1) % ngrp) * 16, 16)
            vn = uidx[pl.ds(gn, 16)]
            tn = (vn >> 7) << 7
            for j in range(16):
                pltpu.make_async_copy(
                    tu_h.at[:, pl.ds(0, 128)], slabs[j], sem).wait()
                extract(slabs[j], l16[j], r0 + j)
                offn = pl.multiple_of(tn[j], 128)
                pltpu.async_copy(tu_h.at[:, pl.ds(offn, 128)], slabs[j], sem)
            return carry

        lax.fori_loop(0, ngrp, grp_body, 0, unroll=False)
        for j in range(16):
            pltpu.make_async_copy(
                tu_h.at[:, pl.ds(0, 128)], slabs[j], sem).wait()
        for f in range(D_USER):
            pltpu.sync_copy(vals.at[pl.ds(f * bpw, bpw)],
                            out_h.at[f, pl.ds(base, bpw)])

    return gather_kernel(user_id, emb_user_t)


def _tc_rest_body(cid_ref, did_ref, dp_ref, tc_ref, td_ref, w_ref,
                  acc_ref, ssq_ref):
    dp = dp_ref[...]
    lanes = lax.broadcasted_iota(jnp.int32, (1, V_SMALL), 1)
    onehot_c = (cid_ref[...] == lanes).astype(jnp.float32)
    onehot_d = (did_ref[...] == lanes).astype(jnp.float32)
    ec = jnp.dot(onehot_c, tc_ref[...], preferred_element_type=jnp.float32)
    ed = jnp.dot(onehot_d, td_ref[...], preferred_element_type=jnp.float32)
    ssq_ref[...] = (jnp.sum(ec * ec, axis=1, keepdims=True)
                    + jnp.sum(ed * ed, axis=1, keepdims=True)
                    + jnp.sum(dp * dp, axis=1, keepdims=True))
    acc = jnp.dot(ec, w_ref[D_USER:D_USER + D_COUNTRY, :],
                  preferred_element_type=jnp.float32)
    acc += jnp.dot(ed, w_ref[D_USER + D_COUNTRY:D_USER + D_COUNTRY + D_DEVICE, :],
                   preferred_element_type=jnp.float32)
    acc += jnp.dot(dp, w_ref[TOTAL - D_DENSE:TOTAL, :],
                   preferred_element_type=jnp.float32)
    acc_ref[...] = acc


def _tc_rest(cid, did, dp, tbl_c, tbl_d, w):
    blk = 4096
    grid = (B // blk,)
    return pl.pallas_call(
        _tc_rest_body,
        grid=grid,
        in_specs=[
            pl.BlockSpec((blk, 1), lambda i: (i, 0)),
            pl.BlockSpec((blk, 1), lambda i: (i, 0)),
            pl.BlockSpec((blk, D_DENSE), lambda i: (i, 0)),
            pl.BlockSpec((V_SMALL, D_COUNTRY), lambda i: (0, 0)),
            pl.BlockSpec((V_SMALL, D_DEVICE), lambda i: (0, 0)),
            pl.BlockSpec((TOTAL, OUT_D), lambda i: (0, 0)),
        ],
        out_specs=[
            pl.BlockSpec((blk, OUT_D), lambda i: (i, 0)),
            pl.BlockSpec((blk, 1), lambda i: (i, 0)),
        ],
        out_shape=[
            jax.ShapeDtypeStruct((B, OUT_D), jnp.float32),
            jax.ShapeDtypeStruct((B, 1), jnp.float32),
        ],
    )(cid, did, dp, tbl_c, tbl_d, w)


def _tc_combine_body(eut_ref, accr_ref, ssqr_ref, w_ref, b_ref, out_ref):
    eut = eut_ref[...]
    ones_u = jnp.ones((D_USER, 1), jnp.float32)
    ssq = ssqr_ref[...] + lax.dot_general(
        eut * eut, ones_u, (((0,), (0,)), ((), ())),
        preferred_element_type=jnp.float32)
    scale = lax.rsqrt(ssq * (1.0 / TOTAL) + EPS)
    acc = accr_ref[...] + lax.dot_general(
        eut, w_ref[0:D_USER, :], (((0,), (0,)), ((), ())),
        preferred_element_type=jnp.float32)
    out_ref[...] = scale * acc + b_ref[...]


def _tc_combine(eut, accr, ssqr, w, b):
    blk = 4096
    grid = (B // blk,)
    return pl.pallas_call(
        _tc_combine_body,
        grid=grid,
        in_specs=[
            pl.BlockSpec((D_USER, blk), lambda i: (0, i)),
            pl.BlockSpec((blk, OUT_D), lambda i: (i, 0)),
            pl.BlockSpec((blk, 1), lambda i: (i, 0)),
            pl.BlockSpec((TOTAL, OUT_D), lambda i: (0, 0)),
            pl.BlockSpec((1, OUT_D), lambda i: (0, 0)),
        ],
        out_specs=pl.BlockSpec((blk, OUT_D), lambda i: (i, 0)),
        out_shape=jax.ShapeDtypeStruct((B, OUT_D), jnp.float32),
    )(eut, accr, ssqr, w, b)


def kernel(user_id, country, device, dense_profile, emb_user, emb_country,
           emb_device, rms_weight, W, b):
    eut = _sc_gather_user(user_id.astype(jnp.int32), emb_user.T)
    w_scaled = rms_weight[:, None] * W
    accr, ssqr = _tc_rest(country.astype(jnp.int32).reshape(B, 1),
                          device.astype(jnp.int32).reshape(B, 1),
                          dense_profile, emb_country, emb_device, w_scaled)
    return _tc_combine(eut, accr, ssqr, w_scaled, b.reshape(1, OUT_D))


# speedup vs baseline: 1.1766x; 1.0028x over previous
import functools

import jax
import jax.numpy as jnp
from jax import lax
from jax.experimental import pallas as pl
from jax.experimental.pallas import tpu as pltpu
from jax.experimental.pallas import tpu_sc as plsc

B = 16384
D_USER, D_COUNTRY, D_DEVICE, D_DENSE = 32, 16, 16, 128
V_USER = 1000000
V_SMALL = 1000
TOTAL = D_USER + D_COUNTRY + D_DEVICE + D_DENSE
OUT_D = 128
EPS = 1.1920928955078125e-07


def _sc_gather_user(user_id, emb_user_t):
    info = plsc.get_sparse_core_info()
    nw = info.num_cores * info.num_subcores
    bpw = B // nw
    npe = D_USER * bpw
    mesh = plsc.VectorSubcoreMesh(core_axis_name="c", subcore_axis_name="s")

    nbuf = 16

    @functools.partial(
        pl.kernel,
        out_type=jax.ShapeDtypeStruct((D_USER, B), jnp.float32),
        mesh=mesh,
        compiler_params=pltpu.CompilerParams(needs_layout_passes=False),
        scratch_types=[
            pltpu.VMEM((bpw,), jnp.int32),
            pltpu.VMEM((npe,), jnp.float32),
        ] + [pltpu.VMEM((D_USER, 128), jnp.float32) for _ in range(nbuf)]
          + [pltpu.SemaphoreType.DMA],
    )
    def gather_kernel(uid_h, tu_h, out_h, uidx, vals, *rest):
        slabs, sem = list(rest[:nbuf]), rest[nbuf]
        wid = lax.axis_index("s") * info.num_cores + lax.axis_index("c")
        base = pl.multiple_of(wid * bpw, bpw)
        pltpu.sync_copy(uid_h.at[pl.ds(base, bpw)], uidx)
        iota16 = lax.iota(jnp.int32, 16)

        def extract(slab, lane_scalar, row):
            lane = jnp.full((16,), lane_scalar, jnp.int32)
            x0 = plsc.load_gather(slab, [iota16, lane])
            x1 = plsc.load_gather(slab, [iota16 + 16, lane])
            idx0 = iota16 * bpw + row
            plsc.store_scatter(vals, [idx0], x0)
            plsc.store_scatter(vals, [idx0 + 16 * bpw], x1)

        ngrp = bpw // 16

        def issue_group(g):
            v16 = uidx[pl.ds(pl.multiple_of(g * 16, 16), 16)]
            t16 = (v16 >> 7) << 7
            for j in range(16):
                off = pl.multiple_of(t16[j], 128)
                pltpu.async_copy(tu_h.at[:, pl.ds(off, 128)], slabs[j], sem)

        issue_group(0)

        def grp_body(g, carry):
            r0 = pl.multiple_of(g * 16, 16)
            l16 = uidx[pl.ds(r0, 16)] & 127
            gn = pl.multiple_of(((g + 1) % ngrp) * 16, 16)
            vn = uidx[pl.ds(gn, 16)]
            tn = (vn >> 7) << 7
            for j in range(16):
                pltpu.make_async_copy(
                    tu_h.at[:, pl.ds(0, 128)], slabs[j], sem).wait()
                extract(slabs[j], l16[j], r0 + j)
                offn = pl.multiple_of(tn[j], 128)
                pltpu.async_copy(tu_h.at[:, pl.ds(offn, 128)], slabs[j], sem)
            return carry

        lax.fori_loop(0, ngrp, grp_body, 0, unroll=False)
        for j in range(16):
            pltpu.make_async_copy(
                tu_h.at[:, pl.ds(0, 128)], slabs[j], sem).wait()
        for f in range(D_USER):
            pltpu.sync_copy(vals.at[pl.ds(f * bpw, bpw)],
                            out_h.at[f, pl.ds(base, bpw)])

    return gather_kernel(user_id, emb_user_t)


def _tc_rest_body(cid_ref, did_ref, dp_ref, tc_ref, td_ref, w_ref,
                  acc_ref, ssq_ref):
    dp = dp_ref[...]
    lanes = lax.broadcasted_iota(jnp.int32, (1, V_SMALL), 1)
    onehot_c = (cid_ref[...] == lanes).astype(jnp.float32)
    onehot_d = (did_ref[...] == lanes).astype(jnp.float32)
    ec = jnp.dot(onehot_c, tc_ref[...], preferred_element_type=jnp.float32)
    ed = jnp.dot(onehot_d, td_ref[...], preferred_element_type=jnp.float32)
    ssq_ref[...] = (jnp.sum(ec * ec, axis=1, keepdims=True)
                    + jnp.sum(ed * ed, axis=1, keepdims=True)
                    + jnp.sum(dp * dp, axis=1, keepdims=True))
    acc = jnp.dot(ec, w_ref[D_USER:D_USER + D_COUNTRY, :],
                  preferred_element_type=jnp.float32)
    acc += jnp.dot(ed, w_ref[D_USER + D_COUNTRY:D_USER + D_COUNTRY + D_DEVICE, :],
                   preferred_element_type=jnp.float32)
    acc += jnp.dot(dp, w_ref[TOTAL - D_DENSE:TOTAL, :],
                   preferred_element_type=jnp.float32)
    acc_ref[...] = acc


def _tc_rest(cid, did, dp, tbl_c, tbl_d, w):
    blk = 4096
    grid = (B // blk,)
    return pl.pallas_call(
        _tc_rest_body,
        grid=grid,
        in_specs=[
            pl.BlockSpec((blk, 1), lambda i: (i, 0)),
            pl.BlockSpec((blk, 1), lambda i: (i, 0)),
            pl.BlockSpec((blk, D_DENSE), lambda i: (i, 0)),
            pl.BlockSpec((V_SMALL, D_COUNTRY), lambda i: (0, 0)),
            pl.BlockSpec((V_SMALL, D_DEVICE), lambda i: (0, 0)),
            pl.BlockSpec((TOTAL, OUT_D), lambda i: (0, 0)),
        ],
        out_specs=[
            pl.BlockSpec((blk, OUT_D), lambda i: (i, 0)),
            pl.BlockSpec((blk, 1), lambda i: (i, 0)),
        ],
        out_shape=[
            jax.ShapeDtypeStruct((B, OUT_D), jnp.float32),
            jax.ShapeDtypeStruct((B, 1), jnp.float32),
        ],
    )(cid, did, dp, tbl_c, tbl_d, w)


def _tc_combine_body(eut_ref, accr_ref, ssqr_ref, w_ref, b_ref, out_ref):
    eut = eut_ref[...]
    ones_u = jnp.ones((D_USER, 1), jnp.float32)
    ssq = ssqr_ref[...] + lax.dot_general(
        eut * eut, ones_u, (((0,), (0,)), ((), ())),
        preferred_element_type=jnp.float32)
    scale = lax.rsqrt(ssq * (1.0 / TOTAL) + EPS)
    acc = accr_ref[...] + lax.dot_general(
        eut, w_ref[0:D_USER, :], (((0,), (0,)), ((), ())),
        preferred_element_type=jnp.float32)
    out_ref[...] = scale * acc + b_ref[...]


def _tc_combine(eut, accr, ssqr, w, b):
    blk = 4096
    grid = (B // blk,)
    return pl.pallas_call(
        _tc_combine_body,
        grid=grid,
        in_specs=[
            pl.BlockSpec((D_USER, blk), lambda i: (0, i)),
            pl.BlockSpec((blk, OUT_D), lambda i: (i, 0)),
            pl.BlockSpec((blk, 1), lambda i: (i, 0)),
            pl.BlockSpec((TOTAL, OUT_D), lambda i: (0, 0)),
            pl.BlockSpec((1, OUT_D), lambda i: (0, 0)),
        ],
        out_specs=pl.BlockSpec((blk, OUT_D), lambda i: (i, 0)),
        out_shape=jax.ShapeDtypeStruct((B, OUT_D), jnp.float32),
    )(eut, accr, ssqr, w, b)


def kernel(user_id, country, device, dense_profile, emb_user, emb_country,
           emb_device, rms_weight, W, b):
    eut = _sc_gather_user(user_id.astype(jnp.int32), emb_user.T)
    w_scaled = rms_weight[:, None] * W
    accr, ssqr = _tc_rest(country.astype(jnp.int32).reshape(B, 1),
                          device.astype(jnp.int32).reshape(B, 1),
                          dense_profile, emb_country, emb_device, w_scaled)
    return _tc_combine(eut, accr, ssqr, w_scaled, b.reshape(1, OUT_D))
